# baseline (speedup 1.0000x reference)
import functools

import jax
import jax.numpy as jnp
from jax import lax
from jax.experimental import pallas as pl
from jax.experimental.pallas import tpu as pltpu
from jax.experimental.pallas import tpu_sc as plsc

B, S, V = 8, 32, 100000
ROWS = B * S
LANES = 16
NUM_WORKERS = 32
ROWS_PER_WORKER = ROWS // NUM_WORKERS
NUM_GROUPS = V // LANES


def _prevfloat_pos(f):
    return plsc.bitcast(plsc.bitcast(f, jnp.int32) - 1, jnp.float32)


def _sc_body(x_hbm, lab_hbm, out_hbm, row_v, lab_v, res_v):
    cid = lax.axis_index("c")
    sid = lax.axis_index("s")
    wid = sid * 2 + cid
    base = wid * ROWS_PER_WORKER
    lane = lax.iota(jnp.int32, LANES)
    pltpu.sync_copy(lab_hbm, lab_v)
    acc = jnp.float32(0.0)
    for r in range(ROWS_PER_WORKER):
        row = base + r
        pltpu.sync_copy(x_hbm.at[row], row_v)
        g0 = row // LANES
        rl = row - g0 * LANES
        lvec = lab_v[pl.ds(g0 * LANES, LANES)]
        lab = jnp.sum(jnp.where(lane == rl, lvec, 0))
        g_lab = lab // LANES
        rloc = lab - g_lab * LANES
        ab = row_v[pl.ds(g_lab * LANES, LANES)].astype(jnp.int32)
        vi = jnp.broadcast_to(jnp.sum(jnp.where(lane == rloc, ab, 0)), (LANES,))
        c1 = vi.astype(jnp.float32)
        t_ge = jnp.where(vi > 0, _prevfloat_pos(c1), c1 - 1.0)
        c2 = (vi + 1).astype(jnp.float32)
        t_gt = jnp.where(vi + 1 > 0, _prevfloat_pos(c2), c2 - 1.0)
        mb = (ab > vi) | ((ab == vi) & (lane < rloc))
        cnt0 = mb.astype(jnp.int32)

        def body_lo(g, cc):
            return cc + (row_v[pl.ds(g, LANES)] > t_ge).astype(jnp.int32)

        def body_hi(g, cc):
            return cc + (row_v[pl.ds(g, LANES)] > t_gt).astype(jnp.int32)

        zeros = jnp.zeros((LANES,), jnp.int32)
        cnt_lo = plsc.parallel_loop(
            0, g_lab * LANES, LANES, unroll=16, carry=zeros)(body_lo)
        cnt_hi = plsc.parallel_loop(
            (g_lab + 1) * LANES, V, LANES, unroll=16, carry=zeros)(body_hi)
        total = jnp.sum(cnt0 + cnt_lo + cnt_hi)
        acc = acc + jnp.where(total <= 4, jnp.float32(1.0 / ROWS), jnp.float32(0.0))
    res_v[...] = jnp.broadcast_to(acc, (LANES,))
    pltpu.sync_copy(res_v, out_hbm.at[wid])


_sc_count = functools.partial(
    pl.kernel,
    out_type=jax.ShapeDtypeStruct((NUM_WORKERS, LANES), jnp.float32),
    mesh=plsc.VectorSubcoreMesh(core_axis_name="c", subcore_axis_name="s"),
    scratch_types=[
        pltpu.VMEM((V,), jnp.float32),
        pltpu.VMEM((ROWS,), jnp.int32),
        pltpu.VMEM((LANES,), jnp.float32),
    ],
    compiler_params=pltpu.CompilerParams(needs_layout_passes=False),
)(_sc_body)


def _tc_combine(p_ref, o_ref):
    o_ref[0, 0] = jnp.sum(p_ref[...]) * jnp.float32(1.0 / LANES)


def _tc_copy(x_ref, o_ref, sem):
    pltpu.async_copy(x_ref, o_ref, sem).wait()


def kernel(y_true, y_pred):
    labels = y_true.astype(jnp.int32).reshape(ROWS)
    x = y_pred.reshape(ROWS, V)
    partials = _sc_count(x, labels)
    y_out = pl.pallas_call(
        _tc_copy,
        in_specs=[pl.BlockSpec(memory_space=pl.ANY)],
        out_specs=pl.BlockSpec(memory_space=pl.ANY),
        scratch_shapes=[pltpu.SemaphoreType.DMA],
        out_shape=jax.ShapeDtypeStruct((B, S, V), jnp.float32),
    )(y_pred)
    value2d = pl.pallas_call(
        _tc_combine,
        out_shape=jax.ShapeDtypeStruct((1, 1), jnp.float32),
        in_specs=[pl.BlockSpec(memory_space=pltpu.VMEM)],
        out_specs=pl.BlockSpec(memory_space=pltpu.SMEM),
    )(partials)
    return (y_out, value2d.reshape(()))


# speedup vs baseline: 24.5337x; 24.5337x over previous
import functools

import jax
import jax.numpy as jnp
from jax import lax
from jax.experimental import pallas as pl
from jax.experimental.pallas import tpu as pltpu
from jax.experimental.pallas import tpu_sc as plsc

B, S, V = 8, 32, 100000
ROWS = B * S
LANES = 16
NUM_WORKERS = 32
ROWS_PER_WORKER = ROWS // NUM_WORKERS
NUM_GROUPS = V // LANES


def _prevfloat_pos(f):
    return plsc.bitcast(plsc.bitcast(f, jnp.int32) - 1, jnp.float32)


def _sc_body(x_hbm, lab_hbm, out_hbm, row_v, lab_v, res_v):
    cid = lax.axis_index("c")
    sid = lax.axis_index("s")
    wid = sid * 2 + cid
    base = wid * ROWS_PER_WORKER
    lane = lax.iota(jnp.int32, LANES)
    pltpu.sync_copy(lab_hbm, lab_v)
    acc = jnp.float32(0.0)
    for r in range(ROWS_PER_WORKER):
        row = base + r
        pltpu.sync_copy(x_hbm.at[row], row_v)
        g0 = row // LANES
        rl = row - g0 * LANES
        lvec = lab_v[pl.ds(g0 * LANES, LANES)]
        lab = jnp.sum(jnp.where(lane == rl, lvec, 0))
        g_lab = lab // LANES
        rloc = lab - g_lab * LANES
        ab = row_v[pl.ds(g_lab * LANES, LANES)].astype(jnp.int32)
        vi = jnp.broadcast_to(jnp.sum(jnp.where(lane == rloc, ab, 0)), (LANES,))
        c1 = vi.astype(jnp.float32)
        t_ge = jnp.where(vi > 0, _prevfloat_pos(c1), c1 - 1.0)
        c2 = (vi + 1).astype(jnp.float32)
        t_gt = jnp.where(vi + 1 > 0, _prevfloat_pos(c2), c2 - 1.0)
        mb = (ab > vi) | ((ab == vi) & (lane < rloc))
        cnt0 = mb.astype(jnp.int32)

        def body_lo(g, cc):
            return cc + (row_v[pl.ds(g, LANES)] > t_ge).astype(jnp.int32)

        def body_hi(g, cc):
            return cc + (row_v[pl.ds(g, LANES)] > t_gt).astype(jnp.int32)

        zeros = jnp.zeros((LANES,), jnp.int32)
        cnt_lo = plsc.parallel_loop(
            0, g_lab * LANES, LANES, unroll=16, carry=zeros)(body_lo)
        cnt_hi = plsc.parallel_loop(
            (g_lab + 1) * LANES, V, LANES, unroll=16, carry=zeros)(body_hi)
        total = jnp.sum(cnt0 + cnt_lo + cnt_hi)
        acc = acc + jnp.where(total <= 4, jnp.float32(1.0 / ROWS), jnp.float32(0.0))
    res_v[...] = jnp.broadcast_to(acc, (LANES,))
    pltpu.sync_copy(res_v, out_hbm.at[wid])


_sc_count = functools.partial(
    pl.kernel,
    out_type=jax.ShapeDtypeStruct((NUM_WORKERS, LANES), jnp.float32),
    mesh=plsc.VectorSubcoreMesh(core_axis_name="c", subcore_axis_name="s"),
    scratch_types=[
        pltpu.VMEM((V,), jnp.float32),
        pltpu.VMEM((ROWS,), jnp.int32),
        pltpu.VMEM((LANES,), jnp.float32),
    ],
    compiler_params=pltpu.CompilerParams(needs_layout_passes=False),
)(_sc_body)


def _tc_combine(p_ref, o_ref):
    o_ref[0, 0] = jnp.sum(p_ref[...]) * jnp.float32(1.0 / LANES)


def _tc_copy(x_ref, o_ref):
    o_ref[...] = x_ref[...]


def kernel(y_true, y_pred):
    labels = y_true.astype(jnp.int32).reshape(ROWS)
    x = y_pred.reshape(ROWS, V)
    partials = _sc_count(x, labels)
    y_out = pl.pallas_call(
        _tc_copy,
        grid=(B, 2),
        in_specs=[pl.BlockSpec((1, S // 2, V), lambda i, j: (i, j, 0))],
        out_specs=pl.BlockSpec((1, S // 2, V), lambda i, j: (i, j, 0)),
        out_shape=jax.ShapeDtypeStruct((B, S, V), jnp.float32),
    )(y_pred)
    value2d = pl.pallas_call(
        _tc_combine,
        out_shape=jax.ShapeDtypeStruct((1, 1), jnp.float32),
        in_specs=[pl.BlockSpec(memory_space=pltpu.VMEM)],
        out_specs=pl.BlockSpec(memory_space=pltpu.SMEM),
    )(partials)
    return (y_out, value2d.reshape(()))


# speedup vs baseline: 27.2760x; 1.1118x over previous
import functools

import jax
import jax.numpy as jnp
from jax import lax
from jax.experimental import pallas as pl
from jax.experimental.pallas import tpu as pltpu
from jax.experimental.pallas import tpu_sc as plsc

B, S, V = 8, 32, 100000
ROWS = B * S
LANES = 16
NUM_WORKERS = 32
TC_BATCHES = 2
SC_ROWS = (B - TC_BATCHES) * S
ROWS_PER_WORKER = SC_ROWS // NUM_WORKERS
GRID_J = 2
SB = S // GRID_J


def _prevfloat_pos(f):
    return plsc.bitcast(plsc.bitcast(f, jnp.int32) - 1, jnp.float32)


def _sc_body(x_hbm, lab_hbm, out_hbm, row_v, lab_v, res_v):
    cid = lax.axis_index("c")
    sid = lax.axis_index("s")
    wid = sid * 2 + cid
    base = wid * ROWS_PER_WORKER
    lane = lax.iota(jnp.int32, LANES)
    pltpu.sync_copy(lab_hbm, lab_v)
    acc = jnp.float32(0.0)
    for r in range(ROWS_PER_WORKER):
        row = base + r
        pltpu.sync_copy(x_hbm.at[row], row_v)
        g0 = row // LANES
        rl = row - g0 * LANES
        lvec = lab_v[pl.ds(g0 * LANES, LANES)]
        lab = jnp.sum(jnp.where(lane == rl, lvec, 0))
        g_lab = lab // LANES
        rloc = lab - g_lab * LANES
        ab = row_v[pl.ds(g_lab * LANES, LANES)].astype(jnp.int32)
        vi = jnp.broadcast_to(jnp.sum(jnp.where(lane == rloc, ab, 0)), (LANES,))
        c1 = vi.astype(jnp.float32)
        t_ge = jnp.where(vi > 0, _prevfloat_pos(c1), c1 - 1.0)
        c2 = (vi + 1).astype(jnp.float32)
        t_gt = jnp.where(vi + 1 > 0, _prevfloat_pos(c2), c2 - 1.0)
        mb = (ab > vi) | ((ab == vi) & (lane < rloc))
        cnt0 = mb.astype(jnp.int32)

        def body_lo(g, cc):
            return cc + (row_v[pl.ds(g, LANES)] > t_ge).astype(jnp.int32)

        def body_hi(g, cc):
            return cc + (row_v[pl.ds(g, LANES)] > t_gt).astype(jnp.int32)

        zeros = jnp.zeros((LANES,), jnp.int32)
        cnt_lo = plsc.parallel_loop(
            0, g_lab * LANES, LANES, unroll=16, carry=zeros)(body_lo)
        cnt_hi = plsc.parallel_loop(
            (g_lab + 1) * LANES, V, LANES, unroll=16, carry=zeros)(body_hi)
        total = jnp.sum(cnt0 + cnt_lo + cnt_hi)
        acc = acc + jnp.where(total <= 4, jnp.float32(1.0 / ROWS), jnp.float32(0.0))
    res_v[...] = jnp.broadcast_to(acc, (LANES,))
    pltpu.sync_copy(res_v, out_hbm.at[wid])


_sc_count = functools.partial(
    pl.kernel,
    out_type=jax.ShapeDtypeStruct((NUM_WORKERS, LANES), jnp.float32),
    mesh=plsc.VectorSubcoreMesh(core_axis_name="c", subcore_axis_name="s"),
    scratch_types=[
        pltpu.VMEM((V,), jnp.float32),
        pltpu.VMEM((ROWS,), jnp.int32),
        pltpu.VMEM((LANES,), jnp.float32),
    ],
    compiler_params=pltpu.CompilerParams(needs_layout_passes=False),
)(_sc_body)


def _tc_copy_count(lab_ref, x_ref, o_ref, cnt_ref):
    i = pl.program_id(0)
    j = pl.program_id(1)
    o_ref[...] = x_ref[...]

    @pl.when((i == 0) & (j == 0))
    def _():
        cnt_ref[0, 0] = jnp.float32(0.0)

    @pl.when(i >= B - TC_BATCHES)
    def _():
        col = lax.broadcasted_iota(jnp.int32, (SB, V), 1)
        labs = lab_ref[0]
        xi = x_ref[0].astype(jnp.int32)
        vi = jnp.sum(jnp.where(col == labs, xi, 0), axis=1, keepdims=True)
        cnt_gt = jnp.sum((xi > vi).astype(jnp.int32), axis=1)
        cnt_eqb = jnp.sum(((xi == vi) & (col < labs)).astype(jnp.int32), axis=1)
        hits = jnp.sum(jnp.where(cnt_gt + cnt_eqb <= 4,
                                 jnp.float32(1.0 / ROWS), jnp.float32(0.0)))
        cnt_ref[0, 0] += hits


def _tc_combine(p_ref, c_ref, o_ref):
    o_ref[0, 0] = jnp.sum(p_ref[...]) * jnp.float32(1.0 / LANES) + c_ref[0, 0]


def kernel(y_true, y_pred):
    labels = y_true.astype(jnp.int32)
    partials = _sc_count(y_pred.reshape(ROWS, V), labels.reshape(ROWS))
    y_out, tc_cnt = pl.pallas_call(
        _tc_copy_count,
        grid=(B, GRID_J),
        in_specs=[
            pl.BlockSpec((1, SB, 1), lambda i, j: (i, j, 0)),
            pl.BlockSpec((1, SB, V), lambda i, j: (i, j, 0)),
        ],
        out_specs=[
            pl.BlockSpec((1, SB, V), lambda i, j: (i, j, 0)),
            pl.BlockSpec((1, 1), lambda i, j: (0, 0), memory_space=pltpu.SMEM),
        ],
        out_shape=[
            jax.ShapeDtypeStruct((B, S, V), jnp.float32),
            jax.ShapeDtypeStruct((1, 1), jnp.float32),
        ],
    )(labels.reshape(B, S, 1), y_pred)
    value2d = pl.pallas_call(
        _tc_combine,
        out_shape=jax.ShapeDtypeStruct((1, 1), jnp.float32),
        in_specs=[pl.BlockSpec(memory_space=pltpu.VMEM),
                  pl.BlockSpec(memory_space=pltpu.SMEM)],
        out_specs=pl.BlockSpec(memory_space=pltpu.SMEM),
    )(partials, tc_cnt)
    return (y_out, value2d.reshape(()))
